# baseline (speedup 1.0000x reference)
import functools

import jax
import jax.numpy as jnp
from jax import lax
from jax.experimental import pallas as pl
from jax.experimental.pallas import tpu as pltpu
from jax.experimental.pallas import tpu_sc as plsc

E = 8
K = 2
D = 1024
F = 1024
T = 2048
CAP = 30.0
TK = T * K

B = 256
NB = TK // B + E
P = NB * B

NW = 32
_CSUM_C = 512


def _router_body(x_ref, gw_ref, pg_ref, w1_ref, w2_ref,
                 be_ref, xi_ref, x3_ref):
    x = x_ref[...]
    x3_ref[...] = x.reshape(T, 8, 128)
    logits = jnp.dot(x, gw_ref[...], preferred_element_type=jnp.float32)
    logits = jnp.tanh(logits / CAP)
    p = jax.nn.softmax(logits, axis=-1)
    iota = lax.broadcasted_iota(jnp.int32, p.shape, 1)
    m1 = jnp.max(p, axis=-1, keepdims=True)
    i1 = jnp.min(jnp.where(p == m1, iota, E), axis=-1, keepdims=True)
    p2 = jnp.where(iota == i1, -1.0, p)
    m2 = jnp.max(p2, axis=-1, keepdims=True)
    i2 = jnp.min(jnp.where(p2 == m2, iota, E), axis=-1, keepdims=True)
    wsum = m1 + m2
    w1_ref[...] = jnp.broadcast_to(m1 / wsum, (T, 16))
    w2_ref[...] = jnp.broadcast_to(m2 / wsum, (T, 16))

    oh = jnp.concatenate(
        [(iota == i1).astype(jnp.float32), (iota == i2).astype(jnp.float32)],
        axis=0)
    C = _CSUM_C
    ri = lax.broadcasted_iota(jnp.int32, (C, C), 0)
    ci = lax.broadcasted_iota(jnp.int32, (C, C), 1)
    tri = (ci <= ri).astype(jnp.float32)
    carry = jnp.zeros((1, E), jnp.float32)
    parts = []
    for q in range(TK // C):
        blk = lax.slice(oh, (q * C, 0), ((q + 1) * C, E))
        s = jnp.dot(tri, blk, preferred_element_type=jnp.float32) + carry
        parts.append(s)
        carry = lax.slice(s, (C - 1, 0), (C, E))
    incl = jnp.concatenate(parts, axis=0)
    counts = carry.astype(jnp.int32)
    sizes = ((counts + (B - 1)) // B) * B
    sizes_f = sizes.astype(jnp.float32)
    r8 = lax.broadcasted_iota(jnp.int32, (E, E), 0)
    c8 = lax.broadcasted_iota(jnp.int32, (E, E), 1)
    stri = (r8 < c8).astype(jnp.float32)
    starts = jnp.dot(sizes_f, stri, preferred_element_type=jnp.float32)
    rank = jnp.sum(oh * incl, axis=1, keepdims=True) - 1.0
    sstart = jnp.sum(oh * starts, axis=1, keepdims=True)
    pos = (sstart + rank).astype(jnp.int32)
    pg_ref[...] = pos.reshape(TK // 128, 128)

    total = jnp.sum(sizes)
    nab = total // B
    jb = lax.broadcasted_iota(jnp.int32, (NB, 1), 0)
    ge = (jb * B >= starts.astype(jnp.int32)).astype(jnp.int32)
    be = jnp.sum(ge, axis=1, keepdims=True) - 1
    eids = lax.broadcasted_iota(jnp.int32, (1, E), 1)
    last_e = jnp.max(eids * (sizes > 0).astype(jnp.int32))
    active = jb < nab
    be_ref[...] = jnp.where(active, be, last_e)
    xi_ref[...] = jnp.where(active, jb, nab - 1)


def _router(x, gate_w):
    return pl.pallas_call(
        _router_body,
        out_shape=(
            jax.ShapeDtypeStruct((TK // 128, 128), jnp.int32),
            jax.ShapeDtypeStruct((T, 16), jnp.float32),
            jax.ShapeDtypeStruct((T, 16), jnp.float32),
            jax.ShapeDtypeStruct((NB, 1), jnp.int32),
            jax.ShapeDtypeStruct((NB, 1), jnp.int32),
            jax.ShapeDtypeStruct((T, 8, 128), jnp.float32),
        ),
    )(x, gate_w)


_G_ROWS = P // NW
_G_CH = 48
_G_NCH = _G_ROWS // _G_CH
_SEG = P // 16
_SLOTS_W = TK // 16


@functools.cache
def _build_sc_gather():
    @functools.partial(
        pl.kernel,
        mesh=plsc.VectorSubcoreMesh(core_axis_name="c", subcore_axis_name="s"),
        out_type=jax.ShapeDtypeStruct((P, 8, 128), jnp.float32),
        scratch_types=[
            pltpu.VMEM_SHARED((P,), jnp.int32),
            pltpu.VMEM((2, 128), jnp.int32),
            pltpu.VMEM((_SEG,), jnp.int32),
            pltpu.VMEM((_G_ROWS,), jnp.int32),
            pltpu.VMEM((_G_CH, 8, 128), jnp.float32),
            pltpu.VMEM((_G_CH, 8, 128), jnp.float32),
            pltpu.SemaphoreType.DMA,
            pltpu.SemaphoreType.DMA,
            pltpu.SemaphoreType.DMA,
            pltpu.SemaphoreType.DMA,
        ],
    )
    def _sc_gather(x_hbm, pos_hbm, out_hbm, shared, pos_v, tok_v,
                   idx_v, b0, b1, g0, g1, w0, w1):
        sid = lax.axis_index("s")
        wid = sid * 2 + lax.axis_index("c")
        ibase = sid * _SEG
        for g in range(_SEG // 16):
            lane = lax.iota(jnp.int32, 16) + (ibase + g * 16)
            tok_v[pl.ds(g * 16, 16)] = (lane * 37) & (T - 1)
        pltpu.sync_copy(tok_v.at[pl.ds(0, _SEG)], shared.at[pl.ds(ibase, _SEG)])
        plsc.subcore_barrier()
        pltpu.sync_copy(pos_hbm.at[pl.ds(2 * sid, 2)], pos_v)
        for r in range(2):
            for g in range(128 // 16):
                lane = lax.iota(jnp.int32, 16) + ((2 * sid + r) * 128 + g * 16)
                tok_v[pl.ds(g * 16, 16)] = lane & (T - 1)
            pltpu.sync_copy(tok_v.at[pl.ds(0, 128)], shared.at[pos_v.at[r]])
        plsc.subcore_barrier()
        base = wid * _G_ROWS
        pltpu.sync_copy(shared.at[pl.ds(base, _G_ROWS)], idx_v)
        bufs, gsem, wsem = (b0, b1), (g0, g1), (w0, w1)
        cps = [None, None]
        wrs = [None, None]
        for c in range(_G_NCH):
            i = c & 1
            if wrs[i] is not None:
                wrs[i].wait()
            cps[i] = pltpu.async_copy(
                x_hbm.at[idx_v.at[pl.ds(c * _G_CH, _G_CH)]], bufs[i], gsem[i])
            if c >= 1:
                j = 1 - i
                cps[j].wait()
                wrs[j] = pltpu.async_copy(
                    bufs[j], out_hbm.at[pl.ds(base + (c - 1) * _G_CH, _G_CH)],
                    wsem[j])
        last = (_G_NCH - 1) & 1
        cps[last].wait()
        wrs[last] = pltpu.async_copy(
            bufs[last], out_hbm.at[pl.ds(base + (_G_NCH - 1) * _G_CH, _G_CH)],
            wsem[last])
        if _G_NCH >= 2:
            wrs[1 - last].wait()
        wrs[last].wait()

    return _sc_gather


def _ffn_body(be_ref, xi_ref, x_ref, win_ref, wout_ref, out_ref):
    j = pl.program_id(0)

    @pl.when(xi_ref[j, 0] == j)
    def _():
        x = x_ref[...].reshape(B, D).astype(jnp.bfloat16)
        h = jnp.dot(x, win_ref[0].reshape(D, F).astype(jnp.bfloat16),
                    preferred_element_type=jnp.float32)
        h = jax.nn.gelu(h)
        y = jnp.dot(h.astype(jnp.bfloat16),
                    wout_ref[0].astype(jnp.bfloat16),
                    preferred_element_type=jnp.float32)
        out_ref[...] = y.reshape(B, 8, 128)


def _ffn(block_e, xidx, xg, w_in, w_out):
    grid_spec = pltpu.PrefetchScalarGridSpec(
        num_scalar_prefetch=2,
        grid=(NB,),
        in_specs=[
            pl.BlockSpec((B, 8, 128), lambda j, be, xi: (xi[j, 0], 0, 0)),
            pl.BlockSpec((1, 8, 128, F), lambda j, be, xi: (be[j, 0], 0, 0, 0)),
            pl.BlockSpec((1, F, D), lambda j, be, xi: (be[j, 0], 0, 0)),
        ],
        out_specs=pl.BlockSpec((B, 8, 128), lambda j, be, xi: (j, 0, 0)),
    )
    return pl.pallas_call(
        _ffn_body,
        grid_spec=grid_spec,
        out_shape=jax.ShapeDtypeStruct((P, 8, 128), jnp.float32),
    )(block_e, xidx, xg, w_in.reshape(E, 8, 128, F), w_out)


_C_TOKS = T // NW
_C_CH = 16
_C_NCH = _C_TOKS // _C_CH


@functools.cache
def _build_sc_combine():
    @functools.partial(
        pl.kernel,
        mesh=plsc.VectorSubcoreMesh(core_axis_name="c", subcore_axis_name="s"),
        out_type=jax.ShapeDtypeStruct((T, 8, 128), jnp.float32),
        scratch_types=[
            pltpu.VMEM((_C_TOKS,), jnp.int32),
            pltpu.VMEM((_C_TOKS,), jnp.int32),
            pltpu.VMEM((_C_TOKS, 16), jnp.float32),
            pltpu.VMEM((_C_TOKS, 16), jnp.float32),
            pltpu.VMEM((_C_CH, 8, 128), jnp.float32),
            pltpu.VMEM((_C_CH, 8, 128), jnp.float32),
            pltpu.VMEM((_C_CH, 8, 128), jnp.float32),
            pltpu.VMEM((_C_CH, 8, 128), jnp.float32),
            pltpu.SemaphoreType.DMA,
            pltpu.SemaphoreType.DMA,
            pltpu.SemaphoreType.DMA,
            pltpu.SemaphoreType.DMA,
        ],
    )
    def _sc_combine(yg_hbm, pos_hbm, w1_hbm, w2_hbm, out_hbm,
                    i1_v, i2_v, w1_v, w2_v, r1a, r2a, r1b, r2b,
                    ga, gb, wa, wb):
        wid = lax.axis_index("s") * 2 + lax.axis_index("c")
        base = wid * _C_TOKS
        row = wid // 2
        off = (wid % 2) * _C_TOKS
        pltpu.sync_copy(pos_hbm.at[row, pl.ds(off, _C_TOKS)], i1_v)
        pltpu.sync_copy(pos_hbm.at[T // 128 + row, pl.ds(off, _C_TOKS)], i2_v)
        pltpu.sync_copy(w1_hbm.at[pl.ds(base, _C_TOKS)], w1_v)
        pltpu.sync_copy(w2_hbm.at[pl.ds(base, _C_TOKS)], w2_v)
        r1s, r2s, gsem, wsem = (r1a, r1b), (r2a, r2b), (ga, gb), (wa, wb)
        cps = [None, None]
        wrs = [None, None]

        def _combine_chunk(c, r1, r2):
            @plsc.parallel_loop(0, _C_CH, 1, unroll=4)
            def _row(i):
                t = c * _C_CH + i
                wv1 = w1_v[t]
                wv2 = w2_v[t]
                for s in range(8):
                    for g in range(128 // 16):
                        sl = pl.ds(g * 16, 16)
                        r1[i, s, sl] = r1[i, s, sl] * wv1 + r2[i, s, sl] * wv2

        for c in range(_C_NCH):
            i = c & 1
            if wrs[i] is not None:
                wrs[i].wait()
            off = c * _C_CH
            cp1 = pltpu.async_copy(
                yg_hbm.at[i1_v.at[pl.ds(off, _C_CH)]], r1s[i], gsem[i])
            cp2 = pltpu.async_copy(
                yg_hbm.at[i2_v.at[pl.ds(off, _C_CH)]], r2s[i], gsem[i])
            cps[i] = (cp1, cp2)
            if c >= 1:
                j = 1 - i
                cps[j][0].wait()
                cps[j][1].wait()
                _combine_chunk(c - 1, r1s[j], r2s[j])
                wrs[j] = pltpu.async_copy(
                    r1s[j], out_hbm.at[pl.ds(base + (c - 1) * _C_CH, _C_CH)],
                    wsem[j])
        last = (_C_NCH - 1) & 1
        cps[last][0].wait()
        cps[last][1].wait()
        _combine_chunk(_C_NCH - 1, r1s[last], r2s[last])
        wrs[last] = pltpu.async_copy(
            r1s[last], out_hbm.at[pl.ds(base + (_C_NCH - 1) * _C_CH, _C_CH)],
            wsem[last])
        if _C_NCH >= 2:
            wrs[1 - last].wait()
        wrs[last].wait()

    return _sc_combine


def kernel(hidden_states, gate_w, w_in, w_out):
    posgrid, w1b, w2b, block_e, xidx, x3 = _router(hidden_states, gate_w)
    xg = _build_sc_gather()(x3, posgrid)
    yg = _ffn(block_e, xidx, xg, w_in, w_out)
    out3 = _build_sc_combine()(yg, posgrid, w1b, w2b)
    return out3.reshape(T, D)


# speedup vs baseline: 1.0345x; 1.0345x over previous
import functools

import jax
import jax.numpy as jnp
from jax import lax
from jax.experimental import pallas as pl
from jax.experimental.pallas import tpu as pltpu
from jax.experimental.pallas import tpu_sc as plsc

E = 8
K = 2
D = 1024
F = 1024
T = 2048
CAP = 30.0
TK = T * K

B = 256
NB = TK // B + E
P = NB * B

NW = 32
_CSUM_C = 512


def _router_body(x_ref, gw_ref, pg_ref, w1_ref, w2_ref,
                 be_ref, xi_ref, x3_ref):
    x = x_ref[...]
    x3_ref[...] = x.reshape(T, 8, 128)
    logits = jnp.dot(x, gw_ref[...], preferred_element_type=jnp.float32)
    logits = jnp.tanh(logits / CAP)
    p = jax.nn.softmax(logits, axis=-1)
    iota = lax.broadcasted_iota(jnp.int32, p.shape, 1)
    m1 = jnp.max(p, axis=-1, keepdims=True)
    i1 = jnp.min(jnp.where(p == m1, iota, E), axis=-1, keepdims=True)
    p2 = jnp.where(iota == i1, -1.0, p)
    m2 = jnp.max(p2, axis=-1, keepdims=True)
    i2 = jnp.min(jnp.where(p2 == m2, iota, E), axis=-1, keepdims=True)
    wsum = m1 + m2
    w1_ref[...] = jnp.broadcast_to(m1 / wsum, (T, 16))
    w2_ref[...] = jnp.broadcast_to(m2 / wsum, (T, 16))

    oh = jnp.concatenate(
        [(iota == i1).astype(jnp.float32), (iota == i2).astype(jnp.float32)],
        axis=0)
    C = _CSUM_C
    ri = lax.broadcasted_iota(jnp.int32, (C, C), 0)
    ci = lax.broadcasted_iota(jnp.int32, (C, C), 1)
    tri = (ci <= ri).astype(jnp.float32)
    carry = jnp.zeros((1, E), jnp.float32)
    parts = []
    for q in range(TK // C):
        blk = lax.slice(oh, (q * C, 0), ((q + 1) * C, E))
        s = jnp.dot(tri, blk, preferred_element_type=jnp.float32) + carry
        parts.append(s)
        carry = lax.slice(s, (C - 1, 0), (C, E))
    incl = jnp.concatenate(parts, axis=0)
    counts = carry.astype(jnp.int32)
    sizes = ((counts + (B - 1)) // B) * B
    sizes_f = sizes.astype(jnp.float32)
    r8 = lax.broadcasted_iota(jnp.int32, (E, E), 0)
    c8 = lax.broadcasted_iota(jnp.int32, (E, E), 1)
    stri = (r8 < c8).astype(jnp.float32)
    starts = jnp.dot(sizes_f, stri, preferred_element_type=jnp.float32)
    rank = jnp.sum(oh * incl, axis=1, keepdims=True) - 1.0
    sstart = jnp.sum(oh * starts, axis=1, keepdims=True)
    pos = (sstart + rank).astype(jnp.int32)
    pg_ref[...] = pos.reshape(TK // 128, 128)

    total = jnp.sum(sizes)
    nab = total // B
    jb = lax.broadcasted_iota(jnp.int32, (NB, 1), 0)
    ge = (jb * B >= starts.astype(jnp.int32)).astype(jnp.int32)
    be = jnp.sum(ge, axis=1, keepdims=True) - 1
    eids = lax.broadcasted_iota(jnp.int32, (1, E), 1)
    last_e = jnp.max(eids * (sizes > 0).astype(jnp.int32))
    active = jb < nab
    be_ref[...] = jnp.where(active, be, last_e)
    xi_ref[...] = jnp.where(active, jb, nab - 1)


def _router(x, gate_w):
    return pl.pallas_call(
        _router_body,
        out_shape=(
            jax.ShapeDtypeStruct((TK // 128, 128), jnp.int32),
            jax.ShapeDtypeStruct((T, 16), jnp.float32),
            jax.ShapeDtypeStruct((T, 16), jnp.float32),
            jax.ShapeDtypeStruct((NB, 1), jnp.int32),
            jax.ShapeDtypeStruct((NB, 1), jnp.int32),
            jax.ShapeDtypeStruct((T, 8, 128), jnp.float32),
        ),
    )(x, gate_w)


_G_ROWS = P // NW
_G_CH = 48
_G_NCH = _G_ROWS // _G_CH
_SEG = P // 16
_SLOTS_W = TK // 16


@functools.cache
def _build_sc_gather():
    @functools.partial(
        pl.kernel,
        mesh=plsc.VectorSubcoreMesh(core_axis_name="c", subcore_axis_name="s"),
        out_type=jax.ShapeDtypeStruct((P, 8, 128), jnp.float32),
        scratch_types=[
            pltpu.VMEM_SHARED((P,), jnp.int32),
            pltpu.VMEM((2, 128), jnp.int32),
            pltpu.VMEM((_SEG,), jnp.int32),
            pltpu.VMEM((_G_ROWS,), jnp.int32),
            pltpu.VMEM((_G_CH, 8, 128), jnp.float32),
            pltpu.VMEM((_G_CH, 8, 128), jnp.float32),
            pltpu.SemaphoreType.DMA,
            pltpu.SemaphoreType.DMA,
            pltpu.SemaphoreType.DMA,
            pltpu.SemaphoreType.DMA,
        ],
    )
    def _sc_gather(x_hbm, pos_hbm, out_hbm, shared, pos_v, tok_v,
                   idx_v, b0, b1, g0, g1, w0, w1):
        sid = lax.axis_index("s")
        wid = sid * 2 + lax.axis_index("c")
        ibase = sid * _SEG
        for g in range(_SEG // 16):
            lane = lax.iota(jnp.int32, 16) + (ibase + g * 16)
            tok_v[pl.ds(g * 16, 16)] = (lane * 37) & (T - 1)
        pltpu.sync_copy(tok_v.at[pl.ds(0, _SEG)], shared.at[pl.ds(ibase, _SEG)])
        plsc.subcore_barrier()
        pltpu.sync_copy(pos_hbm.at[pl.ds(2 * sid, 2)], pos_v)
        for r in range(2):
            for g in range(128 // 16):
                lane = lax.iota(jnp.int32, 16) + ((2 * sid + r) * 128 + g * 16)
                tok_v[pl.ds(g * 16, 16)] = lane & (T - 1)
            pltpu.sync_copy(tok_v.at[pl.ds(0, 128)], shared.at[pos_v.at[r]])
        plsc.subcore_barrier()
        base = wid * _G_ROWS
        pltpu.sync_copy(shared.at[pl.ds(base, _G_ROWS)], idx_v)
        bufs, gsem, wsem = (b0, b1), (g0, g1), (w0, w1)
        cps = [None, None]
        wrs = [None, None]
        for c in range(_G_NCH):
            i = c & 1
            if wrs[i] is not None:
                wrs[i].wait()
            cps[i] = pltpu.async_copy(
                x_hbm.at[idx_v.at[pl.ds(c * _G_CH, _G_CH)]], bufs[i], gsem[i])
            if c >= 1:
                j = 1 - i
                cps[j].wait()
                wrs[j] = pltpu.async_copy(
                    bufs[j], out_hbm.at[pl.ds(base + (c - 1) * _G_CH, _G_CH)],
                    wsem[j])
        last = (_G_NCH - 1) & 1
        cps[last].wait()
        wrs[last] = pltpu.async_copy(
            bufs[last], out_hbm.at[pl.ds(base + (_G_NCH - 1) * _G_CH, _G_CH)],
            wsem[last])
        if _G_NCH >= 2:
            wrs[1 - last].wait()
        wrs[last].wait()

    return _sc_gather


def _ffn_body(be_ref, xi_ref, x_ref, win_ref, wout_ref, out_ref):
    j = pl.program_id(0)

    @pl.when(xi_ref[j, 0] == j)
    def _():
        x = x_ref[...].reshape(B, D)
        h = jnp.dot(x, win_ref[0].reshape(D, F),
                    preferred_element_type=jnp.float32)
        h = jax.nn.gelu(h)
        y = jnp.dot(h, wout_ref[0], preferred_element_type=jnp.float32)
        out_ref[...] = y.reshape(B, 8, 128)


def _ffn(block_e, xidx, xg, w_in, w_out):
    grid_spec = pltpu.PrefetchScalarGridSpec(
        num_scalar_prefetch=2,
        grid=(NB,),
        in_specs=[
            pl.BlockSpec((B, 8, 128), lambda j, be, xi: (xi[j, 0], 0, 0)),
            pl.BlockSpec((1, 8, 128, F), lambda j, be, xi: (be[j, 0], 0, 0, 0)),
            pl.BlockSpec((1, F, D), lambda j, be, xi: (be[j, 0], 0, 0)),
        ],
        out_specs=pl.BlockSpec((B, 8, 128), lambda j, be, xi: (j, 0, 0)),
    )
    return pl.pallas_call(
        _ffn_body,
        grid_spec=grid_spec,
        out_shape=jax.ShapeDtypeStruct((P, 8, 128), jnp.float32),
    )(block_e, xidx, xg, w_in.reshape(E, 8, 128, F), w_out)


_C_TOKS = T // NW
_C_CH = 16
_C_NCH = _C_TOKS // _C_CH


@functools.cache
def _build_sc_combine():
    @functools.partial(
        pl.kernel,
        mesh=plsc.VectorSubcoreMesh(core_axis_name="c", subcore_axis_name="s"),
        out_type=jax.ShapeDtypeStruct((T, 8, 128), jnp.float32),
        scratch_types=[
            pltpu.VMEM((_C_TOKS,), jnp.int32),
            pltpu.VMEM((_C_TOKS,), jnp.int32),
            pltpu.VMEM((_C_TOKS, 16), jnp.float32),
            pltpu.VMEM((_C_TOKS, 16), jnp.float32),
            pltpu.VMEM((_C_CH, 8, 128), jnp.float32),
            pltpu.VMEM((_C_CH, 8, 128), jnp.float32),
            pltpu.VMEM((_C_CH, 8, 128), jnp.float32),
            pltpu.VMEM((_C_CH, 8, 128), jnp.float32),
            pltpu.SemaphoreType.DMA,
            pltpu.SemaphoreType.DMA,
            pltpu.SemaphoreType.DMA,
            pltpu.SemaphoreType.DMA,
        ],
    )
    def _sc_combine(yg_hbm, pos_hbm, w1_hbm, w2_hbm, out_hbm,
                    i1_v, i2_v, w1_v, w2_v, r1a, r2a, r1b, r2b,
                    ga, gb, wa, wb):
        wid = lax.axis_index("s") * 2 + lax.axis_index("c")
        base = wid * _C_TOKS
        row = wid // 2
        off = (wid % 2) * _C_TOKS
        pltpu.sync_copy(pos_hbm.at[row, pl.ds(off, _C_TOKS)], i1_v)
        pltpu.sync_copy(pos_hbm.at[T // 128 + row, pl.ds(off, _C_TOKS)], i2_v)
        pltpu.sync_copy(w1_hbm.at[pl.ds(base, _C_TOKS)], w1_v)
        pltpu.sync_copy(w2_hbm.at[pl.ds(base, _C_TOKS)], w2_v)
        r1s, r2s, gsem, wsem = (r1a, r1b), (r2a, r2b), (ga, gb), (wa, wb)
        cps = [None, None]
        wrs = [None, None]

        def _combine_chunk(c, r1, r2):
            @plsc.parallel_loop(0, _C_CH, 1, unroll=2)
            def _row(i):
                t = c * _C_CH + i
                wv1 = w1_v[t]
                wv2 = w2_v[t]
                for s in range(8):
                    for g in range(128 // 16):
                        sl = pl.ds(g * 16, 16)
                        r1[i, s, sl] = r1[i, s, sl] * wv1 + r2[i, s, sl] * wv2

        for c in range(_C_NCH):
            i = c & 1
            if wrs[i] is not None:
                wrs[i].wait()
            off = c * _C_CH
            cp1 = pltpu.async_copy(
                yg_hbm.at[i1_v.at[pl.ds(off, _C_CH)]], r1s[i], gsem[i])
            cp2 = pltpu.async_copy(
                yg_hbm.at[i2_v.at[pl.ds(off, _C_CH)]], r2s[i], gsem[i])
            cps[i] = (cp1, cp2)
            if c >= 1:
                j = 1 - i
                cps[j][0].wait()
                cps[j][1].wait()
                _combine_chunk(c - 1, r1s[j], r2s[j])
                wrs[j] = pltpu.async_copy(
                    r1s[j], out_hbm.at[pl.ds(base + (c - 1) * _C_CH, _C_CH)],
                    wsem[j])
        last = (_C_NCH - 1) & 1
        cps[last][0].wait()
        cps[last][1].wait()
        _combine_chunk(_C_NCH - 1, r1s[last], r2s[last])
        wrs[last] = pltpu.async_copy(
            r1s[last], out_hbm.at[pl.ds(base + (_C_NCH - 1) * _C_CH, _C_CH)],
            wsem[last])
        if _C_NCH >= 2:
            wrs[1 - last].wait()
        wrs[last].wait()

    return _sc_combine


def kernel(hidden_states, gate_w, w_in, w_out):
    posgrid, w1b, w2b, block_e, xidx, x3 = _router(hidden_states, gate_w)
    xg = _build_sc_gather()(x3, posgrid)
    yg = _ffn(block_e, xidx, xg, w_in, w_out)
    out3 = _build_sc_combine()(yg, posgrid, w1b, w2b)
    return out3.reshape(T, D)


# speedup vs baseline: 1.0414x; 1.0067x over previous
import functools

import jax
import jax.numpy as jnp
from jax import lax
from jax.experimental import pallas as pl
from jax.experimental.pallas import tpu as pltpu
from jax.experimental.pallas import tpu_sc as plsc

E = 8
K = 2
D = 1024
F = 1024
T = 2048
CAP = 30.0
TK = T * K

B = 256
NB = TK // B + E
P = NB * B

NW = 32
_CSUM_C = 512


def _router_body(x_ref, gw_ref, pg_ref, w1_ref, w2_ref,
                 be_ref, xi_ref, x3_ref):
    x = x_ref[...]
    x3_ref[...] = x.reshape(T, 8, 128)
    logits = jnp.dot(x, gw_ref[...], preferred_element_type=jnp.float32)
    logits = jnp.tanh(logits / CAP)
    p = jax.nn.softmax(logits, axis=-1)
    iota = lax.broadcasted_iota(jnp.int32, p.shape, 1)
    m1 = jnp.max(p, axis=-1, keepdims=True)
    i1 = jnp.min(jnp.where(p == m1, iota, E), axis=-1, keepdims=True)
    p2 = jnp.where(iota == i1, -1.0, p)
    m2 = jnp.max(p2, axis=-1, keepdims=True)
    i2 = jnp.min(jnp.where(p2 == m2, iota, E), axis=-1, keepdims=True)
    wsum = m1 + m2
    w1_ref[...] = jnp.broadcast_to(m1 / wsum, (T, 16))
    w2_ref[...] = jnp.broadcast_to(m2 / wsum, (T, 16))

    oh = jnp.concatenate(
        [(iota == i1).astype(jnp.float32), (iota == i2).astype(jnp.float32)],
        axis=0)
    C = _CSUM_C
    ri = lax.broadcasted_iota(jnp.int32, (C, C), 0)
    ci = lax.broadcasted_iota(jnp.int32, (C, C), 1)
    tri = (ci <= ri).astype(jnp.float32)
    carry = jnp.zeros((1, E), jnp.float32)
    parts = []
    for q in range(TK // C):
        blk = lax.slice(oh, (q * C, 0), ((q + 1) * C, E))
        s = jnp.dot(tri, blk, preferred_element_type=jnp.float32) + carry
        parts.append(s)
        carry = lax.slice(s, (C - 1, 0), (C, E))
    incl = jnp.concatenate(parts, axis=0)
    counts = carry.astype(jnp.int32)
    sizes = ((counts + (B - 1)) // B) * B
    sizes_f = sizes.astype(jnp.float32)
    r8 = lax.broadcasted_iota(jnp.int32, (E, E), 0)
    c8 = lax.broadcasted_iota(jnp.int32, (E, E), 1)
    stri = (r8 < c8).astype(jnp.float32)
    starts = jnp.dot(sizes_f, stri, preferred_element_type=jnp.float32)
    rank = jnp.sum(oh * incl, axis=1, keepdims=True) - 1.0
    sstart = jnp.sum(oh * starts, axis=1, keepdims=True)
    pos = (sstart + rank).astype(jnp.int32)
    pg_ref[...] = pos.reshape(TK // 128, 128)

    total = jnp.sum(sizes)
    nab = total // B
    jb = lax.broadcasted_iota(jnp.int32, (NB, 1), 0)
    ge = (jb * B >= starts.astype(jnp.int32)).astype(jnp.int32)
    be = jnp.sum(ge, axis=1, keepdims=True) - 1
    eids = lax.broadcasted_iota(jnp.int32, (1, E), 1)
    last_e = jnp.max(eids * (sizes > 0).astype(jnp.int32))
    active = jb < nab
    be_ref[...] = jnp.where(active, be, last_e)
    xi_ref[...] = jnp.where(active, jb, nab - 1)


def _router(x, gate_w):
    return pl.pallas_call(
        _router_body,
        out_shape=(
            jax.ShapeDtypeStruct((TK // 128, 128), jnp.int32),
            jax.ShapeDtypeStruct((T, 16), jnp.float32),
            jax.ShapeDtypeStruct((T, 16), jnp.float32),
            jax.ShapeDtypeStruct((NB, 1), jnp.int32),
            jax.ShapeDtypeStruct((NB, 1), jnp.int32),
            jax.ShapeDtypeStruct((T, 8, 128), jnp.float32),
        ),
    )(x, gate_w)


_G_ROWS = P // NW
_G_CH = 48
_G_NCH = _G_ROWS // _G_CH
_SEG = P // 16


@functools.cache
def _build_sc_gather():
    @functools.partial(
        pl.kernel,
        mesh=plsc.VectorSubcoreMesh(core_axis_name="c", subcore_axis_name="s"),
        out_type=jax.ShapeDtypeStruct((P, 8, 128), jnp.float32),
        scratch_types=[
            pltpu.VMEM_SHARED((P,), jnp.int32),
            pltpu.VMEM((2, 128), jnp.int32),
            pltpu.VMEM((_SEG,), jnp.int32),
            pltpu.VMEM((_G_ROWS,), jnp.int32),
            pltpu.VMEM((_G_CH, 8, 128), jnp.float32),
            pltpu.VMEM((_G_CH, 8, 128), jnp.float32),
            pltpu.SemaphoreType.DMA,
            pltpu.SemaphoreType.DMA,
            pltpu.SemaphoreType.DMA,
            pltpu.SemaphoreType.DMA,
        ],
    )
    def _sc_gather(x_hbm, pos_hbm, out_hbm, shared, pos_v, tok_v,
                   idx_v, b0, b1, g0, g1, w0, w1):
        sid = lax.axis_index("s")
        wid = sid * 2 + lax.axis_index("c")
        ibase = sid * _SEG
        for g in range(_SEG // 16):
            lane = lax.iota(jnp.int32, 16) + (ibase + g * 16)
            tok_v[pl.ds(g * 16, 16)] = (lane * 37) & (T - 1)
        pltpu.sync_copy(tok_v.at[pl.ds(0, _SEG)], shared.at[pl.ds(ibase, _SEG)])
        plsc.subcore_barrier()
        pltpu.sync_copy(pos_hbm.at[pl.ds(2 * sid, 2)], pos_v)
        for r in range(2):
            for g in range(128 // 16):
                lane = lax.iota(jnp.int32, 16) + ((2 * sid + r) * 128 + g * 16)
                tok_v[pl.ds(g * 16, 16)] = lane & (T - 1)
            pltpu.sync_copy(tok_v.at[pl.ds(0, 128)], shared.at[pos_v.at[r]])
        plsc.subcore_barrier()
        base = wid * _G_ROWS
        pltpu.sync_copy(shared.at[pl.ds(base, _G_ROWS)], idx_v)
        bufs, gsem, wsem = (b0, b1), (g0, g1), (w0, w1)
        cps = [None, None]
        wrs = [None, None]
        for c in range(_G_NCH):
            i = c & 1
            if wrs[i] is not None:
                wrs[i].wait()
            cps[i] = pltpu.async_copy(
                x_hbm.at[idx_v.at[pl.ds(c * _G_CH, _G_CH)]], bufs[i], gsem[i])
            if c >= 1:
                j = 1 - i
                cps[j].wait()
                wrs[j] = pltpu.async_copy(
                    bufs[j], out_hbm.at[pl.ds(base + (c - 1) * _G_CH, _G_CH)],
                    wsem[j])
        last = (_G_NCH - 1) & 1
        cps[last].wait()
        wrs[last] = pltpu.async_copy(
            bufs[last], out_hbm.at[pl.ds(base + (_G_NCH - 1) * _G_CH, _G_CH)],
            wsem[last])
        if _G_NCH >= 2:
            wrs[1 - last].wait()
        wrs[last].wait()

    return _sc_gather


def _ffn_body(be_ref, xi_ref, x_ref, win_ref, wout_ref, out_ref):
    j = pl.program_id(0)

    @pl.when(xi_ref[j, 0] == j)
    def _():
        x = x_ref[...].reshape(B, D)
        h = jnp.dot(x, win_ref[0].reshape(D, F),
                    preferred_element_type=jnp.float32)
        h = jax.nn.gelu(h)
        y = jnp.dot(h, wout_ref[0], preferred_element_type=jnp.float32)
        out_ref[...] = y.reshape(B, 8, 128)


def _ffn(block_e, xidx, xg, w_in, w_out):
    grid_spec = pltpu.PrefetchScalarGridSpec(
        num_scalar_prefetch=2,
        grid=(NB,),
        in_specs=[
            pl.BlockSpec((B, 8, 128), lambda j, be, xi: (xi[j, 0], 0, 0)),
            pl.BlockSpec((1, 8, 128, F), lambda j, be, xi: (be[j, 0], 0, 0, 0)),
            pl.BlockSpec((1, F, D), lambda j, be, xi: (be[j, 0], 0, 0)),
        ],
        out_specs=pl.BlockSpec((B, 8, 128), lambda j, be, xi: (j, 0, 0)),
    )
    return pl.pallas_call(
        _ffn_body,
        grid_spec=grid_spec,
        out_shape=jax.ShapeDtypeStruct((P, 8, 128), jnp.float32),
    )(block_e, xidx, xg, w_in.reshape(E, 8, 128, F), w_out)


_C_TOKS = T // NW
_C_CH = 16
_C_NCH = _C_TOKS // _C_CH


@functools.cache
def _build_sc_combine():
    @functools.partial(
        pl.kernel,
        mesh=plsc.VectorSubcoreMesh(core_axis_name="c", subcore_axis_name="s"),
        out_type=jax.ShapeDtypeStruct((T, 8, 128), jnp.float32),
        scratch_types=[
            pltpu.VMEM((_C_TOKS,), jnp.int32),
            pltpu.VMEM((_C_TOKS,), jnp.int32),
            pltpu.VMEM((_C_TOKS, 16), jnp.float32),
            pltpu.VMEM((_C_TOKS, 16), jnp.float32),
            pltpu.VMEM((_C_CH, 8, 128), jnp.float32),
            pltpu.VMEM((_C_CH, 8, 128), jnp.float32),
            pltpu.VMEM((_C_CH, 8, 128), jnp.float32),
            pltpu.VMEM((_C_CH, 8, 128), jnp.float32),
            pltpu.SemaphoreType.DMA,
            pltpu.SemaphoreType.DMA,
            pltpu.SemaphoreType.DMA,
            pltpu.SemaphoreType.DMA,
        ],
    )
    def _sc_combine(yg_hbm, pos_hbm, w1_hbm, w2_hbm, out_hbm,
                    i1_v, i2_v, w1_v, w2_v, r1a, r2a, r1b, r2b,
                    ga, gb, wa, wb):
        wid = lax.axis_index("s") * 2 + lax.axis_index("c")
        base = wid * _C_TOKS
        row = wid // 2
        off = (wid % 2) * _C_TOKS
        pltpu.sync_copy(pos_hbm.at[row, pl.ds(off, _C_TOKS)], i1_v)
        pltpu.sync_copy(pos_hbm.at[T // 128 + row, pl.ds(off, _C_TOKS)], i2_v)
        pltpu.sync_copy(w1_hbm.at[pl.ds(base, _C_TOKS)], w1_v)
        pltpu.sync_copy(w2_hbm.at[pl.ds(base, _C_TOKS)], w2_v)
        r1s, r2s, gsem, wsem = (r1a, r1b), (r2a, r2b), (ga, gb), (wa, wb)
        cps = [None, None]
        wrs = [None, None]

        def _combine_chunk(c, r1, r2):
            @plsc.parallel_loop(0, _C_CH, 1, unroll=2)
            def _row(i):
                t = c * _C_CH + i
                wv1 = w1_v[t]
                wv2 = w2_v[t]
                for s in range(8):
                    for g in range(128 // 16):
                        sl = pl.ds(g * 16, 16)
                        r1[i, s, sl] = r1[i, s, sl] * wv1 + r2[i, s, sl] * wv2

        for c in range(_C_NCH):
            i = c & 1
            if wrs[i] is not None:
                wrs[i].wait()
            off = c * _C_CH
            cp1 = pltpu.async_copy(
                yg_hbm.at[i1_v.at[pl.ds(off, _C_CH)]], r1s[i], gsem[i])
            cp2 = pltpu.async_copy(
                yg_hbm.at[i2_v.at[pl.ds(off, _C_CH)]], r2s[i], gsem[i])
            cps[i] = (cp1, cp2)
            if c >= 1:
                j = 1 - i
                cps[j][0].wait()
                cps[j][1].wait()
                _combine_chunk(c - 1, r1s[j], r2s[j])
                wrs[j] = pltpu.async_copy(
                    r1s[j], out_hbm.at[pl.ds(base + (c - 1) * _C_CH, _C_CH)],
                    wsem[j])
        last = (_C_NCH - 1) & 1
        cps[last][0].wait()
        cps[last][1].wait()
        _combine_chunk(_C_NCH - 1, r1s[last], r2s[last])
        wrs[last] = pltpu.async_copy(
            r1s[last], out_hbm.at[pl.ds(base + (_C_NCH - 1) * _C_CH, _C_CH)],
            wsem[last])
        if _C_NCH >= 2:
            wrs[1 - last].wait()
        wrs[last].wait()

    return _sc_combine


def kernel(hidden_states, gate_w, w_in, w_out):
    posgrid, w1b, w2b, block_e, xidx, x3 = _router(hidden_states, gate_w)
    xg = _build_sc_gather()(x3, posgrid)
    yg = _ffn(block_e, xidx, xg, w_in, w_out)
    out3 = _build_sc_combine()(yg, posgrid, w1b, w2b)
    return out3.reshape(T, D)
